# baseline (speedup 1.0000x reference)
import functools

import jax
import jax.numpy as jnp
from jax.experimental import pallas as pl

EPS_PD = 1e-6
MARGIN = 0.1


def _loss_kernel(x_ref, out_ref):
    B = x_ref.shape[0]
    D = x_ref.shape[1] // 2
    a_raw = x_ref[:, :D]
    p_raw = x_ref[:, D:]

    ones = jnp.ones((D, 128), dtype=jnp.float32)
    dims = (((1,), (0,)), ((), ()))
    ssp = jax.lax.dot_general(
        p_raw * p_raw, ones, dims, preferred_element_type=jnp.float32)
    invp = jax.lax.rsqrt(jnp.maximum(ssp, 1e-24))
    invp = jnp.concatenate([invp] * (D // 128), axis=1)
    p = p_raw * invp

    ssa = jax.lax.dot_general(
        a_raw * a_raw, ones, dims, preferred_element_type=jnp.float32)
    inva = jax.lax.rsqrt(jnp.maximum(ssa[:, 0:1], 1e-24))

    gdiag = jax.lax.dot_general(
        a_raw * p, ones, dims, preferred_element_type=jnp.float32)

    g = jax.lax.dot_general(
        a_raw, p, (((1,), (1,)), ((), ())),
        preferred_element_type=jnp.float32)

    rows = jax.lax.broadcasted_iota(jnp.int32, (B, B), 0)
    cols = jax.lax.broadcasted_iota(jnp.int32, (B, B), 1)
    masked = jnp.where(rows == cols, jnp.float32(-1e30), g)
    rowmax = jnp.max(masked, axis=1, keepdims=True)

    neg2 = jnp.maximum(2.0 - 2.0 * (inva * rowmax), 0.0)
    pos2 = 2.0 - 2.0 * (inva * gdiag[:, 0:1])

    loss = jnp.mean(jax.nn.relu(pos2 - neg2 + MARGIN))
    out_ref[...] = loss.reshape(1, 1)


@functools.partial(jax.jit)
def kernel(x):
    B = x.shape[0]
    x2 = x.reshape(B, -1)
    out = pl.pallas_call(
        _loss_kernel,
        out_shape=jax.ShapeDtypeStruct((1, 1), jnp.float32),
    )(x2)
    return out[0, 0]


# speedup vs baseline: 1.8248x; 1.8248x over previous
import functools

import jax
import jax.numpy as jnp
from jax.experimental import pallas as pl
from jax.experimental.pallas import tpu as pltpu

EPS_PD = 1e-6
MARGIN = 0.1


def _loss_kernel(x_hbm, out_ref, a_vmem, p_vmem, sem_a, sem_p):
    cp_a = pltpu.make_async_copy(x_hbm.at[:, 0, :], a_vmem, sem_a)
    cp_p = pltpu.make_async_copy(x_hbm.at[:, 1, :], p_vmem, sem_p)
    cp_a.start()
    cp_p.start()
    cp_a.wait()
    cp_p.wait()
    a_raw = a_vmem[...]
    p_raw = p_vmem[...]
    B = a_raw.shape[0]
    D = a_raw.shape[1]

    ones = jnp.ones((D, 128), dtype=jnp.float32)
    dims = (((1,), (0,)), ((), ()))
    ssp = jax.lax.dot_general(
        p_raw * p_raw, ones, dims, preferred_element_type=jnp.float32)
    invp = jax.lax.rsqrt(jnp.maximum(ssp, 1e-24))
    invp = jnp.concatenate([invp] * (D // 128), axis=1)
    p = p_raw * invp

    ssa = jax.lax.dot_general(
        a_raw * a_raw, ones, dims, preferred_element_type=jnp.float32)
    inva = jax.lax.rsqrt(jnp.maximum(ssa[:, 0:1], 1e-24))

    gdiag = jax.lax.dot_general(
        a_raw * p, ones, dims, preferred_element_type=jnp.float32)

    g = jax.lax.dot_general(
        a_raw, p, (((1,), (1,)), ((), ())),
        preferred_element_type=jnp.float32)

    rows = jax.lax.broadcasted_iota(jnp.int32, (B, B), 0)
    cols = jax.lax.broadcasted_iota(jnp.int32, (B, B), 1)
    masked = jnp.where(rows == cols, jnp.float32(-1e30), g)
    rowmax = jnp.max(masked, axis=1, keepdims=True)

    neg2 = jnp.maximum(2.0 - 2.0 * (inva * rowmax), 0.0)
    pos2 = 2.0 - 2.0 * (inva * gdiag[:, 0:1])

    loss = jnp.mean(jax.nn.relu(pos2 - neg2 + MARGIN))
    out_ref[...] = loss.reshape(1, 1)


@functools.partial(jax.jit)
def kernel(x):
    B, _, D = x.shape
    out = pl.pallas_call(
        _loss_kernel,
        in_specs=[pl.BlockSpec(memory_space=pltpu.MemorySpace.HBM)],
        out_specs=pl.BlockSpec(memory_space=pltpu.MemorySpace.VMEM),
        out_shape=jax.ShapeDtypeStruct((1, 1), jnp.float32),
        scratch_shapes=[
            pltpu.VMEM((B, D), jnp.float32),
            pltpu.VMEM((B, D), jnp.float32),
            pltpu.SemaphoreType.DMA,
            pltpu.SemaphoreType.DMA,
        ],
    )(x)
    return out[0, 0]


# speedup vs baseline: 1.8476x; 1.0125x over previous
import functools

import jax
import jax.numpy as jnp
from jax.experimental import pallas as pl
from jax.experimental.pallas import tpu as pltpu

EPS_PD = 1e-6
MARGIN = 0.1


def _loss_kernel(x_hbm, out_ref, a_vmem, p_vmem, sem_a, sem_p):
    cp_p = pltpu.make_async_copy(x_hbm.at[:, 1, :], p_vmem, sem_p)
    cp_a = pltpu.make_async_copy(x_hbm.at[:, 0, :], a_vmem, sem_a)
    cp_p.start()
    cp_a.start()
    cp_p.wait()
    p_raw = p_vmem[...]
    B = p_raw.shape[0]
    D = p_raw.shape[1]

    ones = jnp.ones((D, 128), dtype=jnp.float32)
    dims = (((1,), (0,)), ((), ()))
    ssp = jax.lax.dot_general(
        p_raw * p_raw, ones, dims, preferred_element_type=jnp.float32)
    invp = jax.lax.rsqrt(jnp.maximum(ssp, 1e-24))
    invp = jnp.concatenate([invp] * (D // 128), axis=1)
    p = p_raw * invp

    cp_a.wait()
    a_raw = a_vmem[...]
    ssa = jax.lax.dot_general(
        a_raw * a_raw, ones, dims, preferred_element_type=jnp.float32)
    inva = jax.lax.rsqrt(jnp.maximum(ssa[:, 0:1], 1e-24))

    gdiag = jax.lax.dot_general(
        a_raw * p, ones, dims, preferred_element_type=jnp.float32)

    g = jax.lax.dot_general(
        a_raw, p, (((1,), (1,)), ((), ())),
        preferred_element_type=jnp.float32)

    rows = jax.lax.broadcasted_iota(jnp.int32, (B, B), 0)
    cols = jax.lax.broadcasted_iota(jnp.int32, (B, B), 1)
    masked = jnp.where(rows == cols, jnp.float32(-1e30), g)
    rowmax = jnp.max(masked, axis=1, keepdims=True)

    neg2 = jnp.maximum(2.0 - 2.0 * (inva * rowmax), 0.0)
    pos2 = 2.0 - 2.0 * (inva * gdiag[:, 0:1])

    loss = jnp.mean(jax.nn.relu(pos2 - neg2 + MARGIN))
    out_ref[...] = loss.reshape(1, 1)


@functools.partial(jax.jit)
def kernel(x):
    B, _, D = x.shape
    out = pl.pallas_call(
        _loss_kernel,
        in_specs=[pl.BlockSpec(memory_space=pltpu.MemorySpace.HBM)],
        out_specs=pl.BlockSpec(memory_space=pltpu.MemorySpace.VMEM),
        out_shape=jax.ShapeDtypeStruct((1, 1), jnp.float32),
        scratch_shapes=[
            pltpu.VMEM((B, D), jnp.float32),
            pltpu.VMEM((B, D), jnp.float32),
            pltpu.SemaphoreType.DMA,
            pltpu.SemaphoreType.DMA,
        ],
    )(x)
    return out[0, 0]
